# speedup vs baseline: 4.6190x; 4.6190x over previous
import functools
import math

import jax
import jax.numpy as jnp
from jax import lax
from jax.experimental import pallas as pl
from jax.experimental.pallas import tpu as pltpu
from jax.experimental.pallas import tpu_sc as plsc

_EMB = 128
_SCALE = math.sqrt(_EMB)

_NC = 2
_NS = 16
_NW = _NC * _NS

_CHUNK = 128


def _scale_body(t_ref, o_ref):
    o_ref[...] = t_ref[...] * _SCALE


def _scale_table(table):
    rows = table.shape[0]
    block = 1000
    return pl.pallas_call(
        _scale_body,
        grid=(rows // block,),
        in_specs=[pl.BlockSpec((block, _EMB), lambda i: (i, 0))],
        out_specs=pl.BlockSpec((block, _EMB), lambda i: (i, 0)),
        out_shape=jax.ShapeDtypeStruct((rows, _EMB), jnp.float32),
    )(table)


def _make_gather(n_idx):
    per_w = n_idx // _NW
    n_chunks = per_w // _CHUNK
    mesh = plsc.VectorSubcoreMesh(core_axis_name="c", subcore_axis_name="s")

    @functools.partial(
        pl.kernel,
        mesh=mesh,
        out_type=jax.ShapeDtypeStruct((n_idx, _EMB), jnp.float32),
        scratch_types=[
            pltpu.VMEM((_CHUNK,), jnp.int32),
            pltpu.VMEM((_CHUNK, _EMB), jnp.float32),
            pltpu.SemaphoreType.DMA,
        ],
    )
    def k(table_hbm, idx_hbm, out_hbm, idx_v, rows_v, sem):
        wid = lax.axis_index("s") * _NC + lax.axis_index("c")
        base = wid * per_w

        def body(g, carry):
            off = base + g * _CHUNK
            pltpu.sync_copy(idx_hbm.at[pl.ds(off, _CHUNK)], idx_v)
            pltpu.async_copy(table_hbm.at[idx_v], rows_v, sem).wait()
            pltpu.sync_copy(rows_v, out_hbm.at[pl.ds(off, _CHUNK)])
            return carry

        lax.fori_loop(0, n_chunks, body, 0)

    return k


def kernel(tokens, table):
    b, l = tokens.shape
    idx = tokens.reshape(b * l)
    table_scaled = _scale_table(table)
    out = _make_gather(b * l)(table_scaled, idx)
    return out.reshape(b, l, _EMB)


# speedup vs baseline: 7.5157x; 1.6271x over previous
import functools
import math

import jax
import jax.numpy as jnp
from jax import lax
from jax.experimental import pallas as pl
from jax.experimental.pallas import tpu as pltpu
from jax.experimental.pallas import tpu_sc as plsc

_EMB = 128
_SCALE = math.sqrt(_EMB)

_NC = 2
_NS = 16
_NW = _NC * _NS

_CHUNK = 128
_NBUF = 4
_LAG = 2


def _scale_body(t_ref, o_ref):
    o_ref[...] = t_ref[...] * _SCALE


def _scale_table(table):
    rows = table.shape[0]
    block = 1000
    return pl.pallas_call(
        _scale_body,
        grid=(rows // block,),
        in_specs=[pl.BlockSpec((block, _EMB), lambda i: (i, 0))],
        out_specs=pl.BlockSpec((block, _EMB), lambda i: (i, 0)),
        out_shape=jax.ShapeDtypeStruct((rows, _EMB), jnp.float32),
    )(table)


def _make_gather(n_idx):
    per_w = n_idx // _NW
    n_chunks = per_w // _CHUNK
    assert per_w * _NW == n_idx and n_chunks * _CHUNK == per_w
    assert n_chunks % _NBUF == 0 and n_chunks >= 2 * _NBUF
    mesh = plsc.VectorSubcoreMesh(core_axis_name="c", subcore_axis_name="s")

    @functools.partial(
        pl.kernel,
        mesh=mesh,
        out_type=jax.ShapeDtypeStruct((n_idx, _EMB), jnp.float32),
        scratch_types=[
            pltpu.VMEM((n_chunks, _CHUNK), jnp.int32),
            pltpu.VMEM((_NBUF, _CHUNK, _EMB), jnp.float32),
        ]
        + [pltpu.SemaphoreType.DMA] * (2 * _NBUF),
    )
    def k(table_hbm, idx_hbm, out_hbm, idx_v, rows_v, *sems):
        sem_g = sems[:_NBUF]
        sem_s = sems[_NBUF:]
        wid = lax.axis_index("s") * _NC + lax.axis_index("c")
        gbase = wid * n_chunks

        pltpu.sync_copy(idx_hbm.at[pl.ds(gbase, n_chunks)], idx_v)

        def gather(t, slot):
            pltpu.async_copy(
                table_hbm.at[idx_v.at[t]], rows_v.at[slot], sem_g[slot])

        def wait_gather(slot):
            pltpu.make_async_copy(
                table_hbm.at[pl.ds(0, _CHUNK)], rows_v.at[slot],
                sem_g[slot]).wait()

        def scatter(t, slot):
            pltpu.async_copy(
                rows_v.at[slot],
                out_hbm.at[pl.ds((gbase + t) * _CHUNK, _CHUNK)], sem_s[slot])

        def wait_scatter(slot):
            pltpu.make_async_copy(
                rows_v.at[slot], out_hbm.at[pl.ds(0, _CHUNK)],
                sem_s[slot]).wait()

        for u in range(_LAG):
            gather(u, u)
        for b in range(_NBUF - _LAG):
            u, s = _LAG + b, b
            gather(u, u)
            wait_gather(s)
            scatter(s, s)
        for b in range(_LAG):
            u, s = _NBUF + b, _NBUF - _LAG + b
            wait_scatter(u % _NBUF)
            gather(u, u % _NBUF)
            wait_gather(s)
            scatter(s, s)

        def body(o, carry):
            for b in range(_NBUF):
                u = _NBUF * o + _LAG + b
                s = _NBUF * o + b
                us = (_LAG + b) % _NBUF
                wait_scatter(us)
                gather(u, us)
                wait_gather(b)
                scatter(s, b)
            return carry

        lax.fori_loop(1, n_chunks // _NBUF - 1, body, 0, unroll=False)

        for b in range(_LAG):
            u = n_chunks - _LAG + b
            wait_scatter(u % _NBUF)
            gather(u, u % _NBUF)
        for b in range(_NBUF):
            s = n_chunks - _NBUF + b
            wait_gather(s % _NBUF)
            scatter(s, s % _NBUF)
        for b in range(_NBUF):
            wait_scatter(b)

    return k


def kernel(tokens, table):
    b, l = tokens.shape
    n_idx = b * l
    idx = tokens.reshape(n_idx // _CHUNK, _CHUNK)
    table_scaled = _scale_table(table)
    out = _make_gather(n_idx)(table_scaled, idx)
    return out.reshape(b, l, _EMB)


# speedup vs baseline: 9.1625x; 1.2191x over previous
import functools
import math

import jax
import jax.numpy as jnp
from jax import lax
from jax.experimental import pallas as pl
from jax.experimental.pallas import tpu as pltpu
from jax.experimental.pallas import tpu_sc as plsc

_EMB = 128
_SCALE = math.sqrt(_EMB)

_NC = 2
_NS = 16
_NW = _NC * _NS

_CHUNK = 128
_NBUF = 4
_LAG = 2


def _make_gather(n_idx):
    per_w = n_idx // _NW
    n_chunks = per_w // _CHUNK
    assert per_w * _NW == n_idx and n_chunks * _CHUNK == per_w
    assert n_chunks % _NBUF == 0 and n_chunks >= 2 * _NBUF
    mesh = plsc.VectorSubcoreMesh(core_axis_name="c", subcore_axis_name="s")

    @functools.partial(
        pl.kernel,
        mesh=mesh,
        out_type=jax.ShapeDtypeStruct((n_idx, _EMB), jnp.float32),
        scratch_types=[
            pltpu.VMEM((n_chunks, _CHUNK), jnp.int32),
            pltpu.VMEM((_NBUF, _CHUNK, _EMB), jnp.float32),
        ]
        + [pltpu.SemaphoreType.DMA] * (2 * _NBUF),
    )
    def k(table_hbm, idx_hbm, out_hbm, idx_v, rows_v, *sems):
        sem_g = sems[:_NBUF]
        sem_s = sems[_NBUF:]
        wid = lax.axis_index("s") * _NC + lax.axis_index("c")
        gbase = wid * n_chunks

        pltpu.sync_copy(idx_hbm.at[pl.ds(gbase, n_chunks)], idx_v)

        def gather(t, slot):
            pltpu.async_copy(
                table_hbm.at[idx_v.at[t]], rows_v.at[slot], sem_g[slot])

        def wait_gather(slot):
            pltpu.make_async_copy(
                table_hbm.at[pl.ds(0, _CHUNK)], rows_v.at[slot],
                sem_g[slot]).wait()

        def scatter(t, slot):
            pltpu.async_copy(
                rows_v.at[slot],
                out_hbm.at[pl.ds((gbase + t) * _CHUNK, _CHUNK)], sem_s[slot])

        def wait_scatter(slot):
            pltpu.make_async_copy(
                rows_v.at[slot], out_hbm.at[pl.ds(0, _CHUNK)],
                sem_s[slot]).wait()

        def scale(slot):
            @plsc.parallel_loop(0, _CHUNK, step=1)
            def _(r):
                for j in range(_EMB // 16):
                    sl = pl.ds(j * 16, 16)
                    rows_v[slot, r, sl] = rows_v[slot, r, sl] * _SCALE

        for u in range(_LAG):
            gather(u, u)
        for b in range(_NBUF - _LAG):
            u, s = _LAG + b, b
            gather(u, u)
            wait_gather(s)
            scale(s)
            scatter(s, s)
        for b in range(_LAG):
            u, s = _NBUF + b, _NBUF - _LAG + b
            wait_scatter(u % _NBUF)
            gather(u, u % _NBUF)
            wait_gather(s)
            scale(s)
            scatter(s, s)

        def body(o, carry):
            for b in range(_NBUF):
                u = _NBUF * o + _LAG + b
                s = _NBUF * o + b
                us = (_LAG + b) % _NBUF
                wait_scatter(us)
                gather(u, us)
                wait_gather(b)
                scale(b)
                scatter(s, b)
            return carry

        lax.fori_loop(1, n_chunks // _NBUF - 1, body, 0, unroll=False)

        for b in range(_LAG):
            u = n_chunks - _LAG + b
            wait_scatter(u % _NBUF)
            gather(u, u % _NBUF)
        for b in range(_NBUF):
            s = n_chunks - _NBUF + b
            wait_gather(s % _NBUF)
            scale(s % _NBUF)
            scatter(s, s % _NBUF)
        for b in range(_NBUF):
            wait_scatter(b)

    return k


def kernel(tokens, table):
    b, l = tokens.shape
    n_idx = b * l
    idx = tokens.reshape(n_idx // _CHUNK, _CHUNK)
    out = _make_gather(n_idx)(table, idx)
    return out.reshape(b, l, _EMB)
